# baseline (speedup 1.0000x reference)
import functools

import jax
import jax.numpy as jnp
from jax import lax
from jax.experimental import pallas as pl
from jax.experimental.pallas import tpu as pltpu
from jax.experimental.pallas import tpu_sc as plsc

BATCH = 16384
EMB = 64
HID = 128
NW = 32
BPW = BATCH // NW
CHUNK = 128


def _sc_gather(domain_id, go_id, W_domain, W_go):
    mesh = plsc.VectorSubcoreMesh(core_axis_name="c", subcore_axis_name="s")

    @functools.partial(
        pl.kernel,
        mesh=mesh,
        out_type=(
            jax.ShapeDtypeStruct((BATCH, EMB), jnp.float32),
            jax.ShapeDtypeStruct((BATCH, EMB), jnp.float32),
        ),
        scratch_types=(
            pltpu.VMEM((BPW,), jnp.int32),
            pltpu.VMEM((BPW,), jnp.int32),
            pltpu.VMEM((BPW, EMB), jnp.float32),
            pltpu.VMEM((BPW, EMB), jnp.float32),
            pltpu.SemaphoreType.DMA,
        ),
        compiler_params=pltpu.CompilerParams(use_tc_tiling_on_sc=False),
    )
    def gather_kernel(dom_hbm, go_hbm, wd_hbm, wg_hbm, outd_hbm, outg_hbm,
                      idx_d, idx_g, rows_d, rows_g, sem):
        wid = lax.axis_index("s") * 2 + lax.axis_index("c")
        base = wid * BPW
        pltpu.sync_copy(dom_hbm.at[pl.ds(base, BPW)], idx_d)
        pltpu.sync_copy(go_hbm.at[pl.ds(base, BPW)], idx_g)
        copies = []
        for c in range(BPW // CHUNK):
            sl = pl.ds(c * CHUNK, CHUNK)
            copies.append(pltpu.async_copy(wd_hbm.at[idx_d.at[sl]], rows_d.at[sl], sem))
            copies.append(pltpu.async_copy(wg_hbm.at[idx_g.at[sl]], rows_g.at[sl], sem))
        for cp in copies:
            cp.wait()
        pltpu.sync_copy(rows_d, outd_hbm.at[pl.ds(base, BPW)])
        pltpu.sync_copy(rows_g, outg_hbm.at[pl.ds(base, BPW)])

    return gather_kernel(domain_id, go_id, W_domain, W_go)


def _head_body(d_ref, g_ref, w1_ref, b1_ref, w2_ref, b2_ref, o_ref):
    d = d_ref[...]
    g = g_ref[...]
    nd = jnp.sqrt(jnp.sum(d * d, axis=1, keepdims=True))
    ng = jnp.sqrt(jnp.sum(g * g, axis=1, keepdims=True))
    sd = jnp.where(nd > 1.0, 1.0 / (nd + 1e-7), 1.0)
    sg = jnp.where(ng > 1.0, 1.0 / (ng + 1e-7), 1.0)
    feat = (d * sd) * (g * sg)
    h = jnp.maximum(
        jnp.dot(feat, w1_ref[...], preferred_element_type=jnp.float32) + b1_ref[...],
        0.0,
    )
    o = jnp.sum(h * w2_ref[...], axis=1) + b2_ref[0, 0]
    o_ref[...] = jax.nn.sigmoid(o)


def _tc_head(d_rows, g_rows, W1, b1, W2, b2, blk=2048):
    nblk = BATCH // blk
    out = pl.pallas_call(
        _head_body,
        grid=(nblk,),
        in_specs=[
            pl.BlockSpec((blk, EMB), lambda i: (i, 0)),
            pl.BlockSpec((blk, EMB), lambda i: (i, 0)),
            pl.BlockSpec((EMB, HID), lambda i: (0, 0)),
            pl.BlockSpec((1, HID), lambda i: (0, 0)),
            pl.BlockSpec((1, HID), lambda i: (0, 0)),
            pl.BlockSpec((1, 1), lambda i: (0, 0)),
        ],
        out_specs=pl.BlockSpec((blk,), lambda i: (i,)),
        out_shape=jax.ShapeDtypeStruct((BATCH,), jnp.float32),
    )(d_rows, g_rows, W1, b1.reshape(1, HID), W2.reshape(1, HID), b2.reshape(1, 1))
    return out


def kernel(domain_id, go_id, W_domain, W_go, W1, b1, W2, b2):
    d_rows, g_rows = _sc_gather(domain_id, go_id, W_domain, W_go)
    return _tc_head(d_rows, g_rows, W1, b1, W2, b2)
